# baseline (speedup 1.0000x reference)
import functools

import jax
import jax.numpy as jnp
from jax import lax
from jax.experimental import pallas as pl
from jax.experimental.pallas import tpu as pltpu
from jax.experimental.pallas import tpu_sc as plsc

N = 10000
NP = 10240
CHN = 256
HF = 128
CS = 128
N_SUB = 16
RPT = NP // N_SUB


@functools.cache
def _sc_agg(compute_deg: bool, nch: int):
    assert nch % 4 == 0 and nch >= 12
    mesh = plsc.VectorSubcoreMesh(core_axis_name="c", subcore_axis_name="s",
                                  num_cores=2, num_subcores=N_SUB)
    out_type = [
        jax.ShapeDtypeStruct((NP, HF), jnp.float32),
        jax.ShapeDtypeStruct((NP, HF), jnp.float32),
    ]
    scratch = (
        [pltpu.VMEM((2, CS), jnp.int32)] * 4
        + [pltpu.VMEM((64, 256), jnp.float32)] * 2
        + [pltpu.VMEM_SHARED((NP, HF), jnp.float32)]
        + [pltpu.SemaphoreType.DMA] * 8
    )
    if compute_deg:
        out_type += [
            jax.ShapeDtypeStruct((NP,), jnp.float32),
            jax.ShapeDtypeStruct((NP,), jnp.float32),
        ]
        scratch += [
            pltpu.VMEM((CS,), jnp.float32),
            pltpu.VMEM((RPT,), jnp.float32),
            pltpu.VMEM_SHARED((NP,), jnp.float32),
        ]

    def body(xl, xr, eidx_h, out_l, out_r, *rest):
        if compute_deg:
            (deg0_out, deg1_out, i0, i1, i2, i3, r0, r1, acc,
             m0, m1, m2, m3, g0, g1, s0, s1, ones_v, zdeg_v, dacc) = rest
        else:
            (i0, i1, i2, i3, r0, r1, acc,
             m0, m1, m2, m3, g0, g1, s0, s1) = rest
        islot = (i0, i1, i2, i3)
        isem = (m0, m1, m2, m3)
        rows = (r0, r1)
        gsem = (g0, g1)
        ssem = (s0, s1)
        cid = lax.axis_index("c")
        sid = lax.axis_index("s")
        base = sid * RPT

        def zrow(i, _):
            r0[i // 16, pl.ds((i % 16) * 16, 16)] = (
                jnp.zeros((16,), jnp.float32))
            return 0
        lax.fori_loop(0, 64 * 16, zrow, 0)

        if compute_deg:
            def fill_ones(i, _):
                ones_v[pl.ds(i * 16, 16)] = jnp.ones((16,), jnp.float32)
                return 0
            lax.fori_loop(0, CS // 16, fill_ones, 0)

            def zdeg(i, _):
                zdeg_v[pl.ds(i * 16, 16)] = jnp.zeros((16,), jnp.float32)
                return 0
            lax.fori_loop(0, RPT // 16, zdeg, 0)
            pltpu.sync_copy(zdeg_v, dacc.at[pl.ds(base, RPT)])

        plsc.subcore_barrier()

        def run_core(tbl, deg_par):
            def ifire(j, k):
                pltpu.async_copy(eidx_h.at[sid, j], islot[k], isem[k])

            def iwait(j, k):
                pltpu.make_async_copy(eidx_h.at[sid, j], islot[k],
                                      isem[k]).wait()

            def gfire(j, b, k):
                return
                pltpu.async_copy(tbl.at[islot[k].at[0]], rows[b], gsem[b])

            def gwait(j, b, k):
                return
                pltpu.make_async_copy(tbl.at[islot[k].at[0]], rows[b],
                                      gsem[b]).wait()

            def p5(j, _):
                pltpu.sync_copy(eidx_h.at[sid, j], islot[0])
                for h in range(2):
                    pltpu.async_copy(
                        tbl.at[islot[0].at[0, pl.ds(h * 64, 64)]],
                        rows[h], gsem[h])
                for h in range(2):
                    pltpu.make_async_copy(
                        tbl.at[islot[0].at[0, pl.ds(h * 64, 64)]],
                        rows[h], gsem[h]).wait()
                return 0
            lax.fori_loop(0, nch, p5, 0)

            def sfire(j, b, k):
                pltpu.async_copy(rows[b], acc.at[islot[k].at[1]], ssem[b],
                                 add=True)

            def swait(j, b, k):
                pltpu.make_async_copy(rows[b], acc.at[islot[k].at[1]],
                                      ssem[b]).wait()

            def step(j, b, k, fire_i=True, fire_g=True, first=False):
                if fire_g:
                    iwait(j + 1, (k + 1) % 4)
                if not first:
                    swait(j - 1, 1 - b, (k - 1) % 4)
                if fire_g:
                    gfire(j + 1, 1 - b, (k + 1) % 4)
                if fire_i:
                    ifire(j + 3, (k + 3) % 4)
                gwait(j, b, k)
                sfire(j, b, k)
                if compute_deg and b == deg_par:
                    pltpu.sync_copy(ones_v, dacc.at[islot[k].at[1]], add=True)

            if False:
                ifire(0, 0)
                ifire(1, 1)
                ifire(2, 2)
                iwait(0, 0)
                gfire(0, 0, 0)
                step(0, 0, 0, first=True)
                step(1, 1, 1)
                step(2, 0, 2)
                step(3, 1, 3)

                def obody(o, _):
                    j0 = o * 4 + 4
                    for t in range(4):
                        step(j0 + t, t % 2, t)
                    return 0
                lax.fori_loop(0, (nch - 8) // 4, obody, 0)

                step(nch - 4, 0, 0)
                step(nch - 3, 1, 1, fire_i=False)
                step(nch - 2, 0, 2, fire_i=False)
                step(nch - 1, 1, 3, fire_i=False, fire_g=False)
                swait(nch - 1, 1, 3)

        @pl.when(cid == 0)
        def _():
            run_core(xl, 0)

        @pl.when(cid == 1)
        def _():
            run_core(xr, 1)

        plsc.subcore_barrier()

        @pl.when(cid == 0)
        def _():
            pltpu.sync_copy(acc.at[pl.ds(base, RPT)], out_l.at[pl.ds(base, RPT)])
            if compute_deg:
                pltpu.sync_copy(dacc.at[pl.ds(base, RPT)],
                                deg0_out.at[pl.ds(base, RPT)])

        @pl.when(cid == 1)
        def _():
            pltpu.sync_copy(acc.at[pl.ds(base, RPT)], out_r.at[pl.ds(base, RPT)])
            if compute_deg:
                pltpu.sync_copy(dacc.at[pl.ds(base, RPT)],
                                deg1_out.at[pl.ds(base, RPT)])

    return pl.kernel(body, out_type=out_type, mesh=mesh, scratch_types=scratch)


@functools.cache
def _tc_layer(relu: bool, split: bool):
    BLK = 1024

    def body(aggl, aggr, xl, xr, deg0, deg1, A, B, C, D, b, *outs):
        d = jnp.maximum(deg0[...] + deg1[...], 1.0)
        ml = aggl[...] / d
        mr = aggr[...] / d
        acc = jnp.dot(ml, A[...], preferred_element_type=jnp.float32)
        acc = acc + jnp.dot(mr, B[...], preferred_element_type=jnp.float32)
        acc = acc + jnp.dot(xl[...], C[...], preferred_element_type=jnp.float32)
        acc = acc + jnp.dot(xr[...], D[...], preferred_element_type=jnp.float32)
        acc = acc + b[...]
        if relu:
            acc = jnp.maximum(acc, 0.0)
        if split:
            outs[0][...] = acc[:, :HF]
            outs[1][...] = acc[:, HF:]
        else:
            outs[0][...] = acc

    row = lambda i: (i, 0)
    full = lambda i: (0, 0)
    in_specs = (
        [pl.BlockSpec((BLK, HF), row)] * 4
        + [pl.BlockSpec((BLK, 1), row)] * 2
        + [pl.BlockSpec((HF, CHN), full)] * 4
        + [pl.BlockSpec((1, CHN), full)]
    )
    if split:
        out_specs = [pl.BlockSpec((BLK, HF), row)] * 2
        out_shape = [jax.ShapeDtypeStruct((NP, HF), jnp.float32)] * 2
    else:
        out_specs = pl.BlockSpec((BLK, CHN), row)
        out_shape = jax.ShapeDtypeStruct((NP, CHN), jnp.float32)
    return pl.pallas_call(body, grid=(NP // BLK,), in_specs=in_specs,
                          out_specs=out_specs, out_shape=out_shape)


def kernel(x, edge_index, W1l, b1l, W1r, W2l, b2l, W2r):
    x = x.astype(jnp.float32)
    src = edge_index[0].astype(jnp.int32)
    dst = edge_index[1].astype(jnp.int32)
    e = src.shape[0]
    nch = -(-e // (N_SUB * CS * 4)) * 4
    pad = nch * N_SUB * CS - e
    srcp = jnp.concatenate(
        [src, jnp.zeros((pad,), jnp.int32)]).reshape(N_SUB, nch, CS)
    dstp = jnp.concatenate(
        [dst, N + (jnp.arange(pad, dtype=jnp.int32) % (NP - N))]
    ).reshape(N_SUB, nch, CS)
    eidx = jnp.stack([srcp, dstp], axis=2)
    xp = jnp.pad(x, ((0, NP - N), (0, 0)))
    xl, xr = xp[:, :HF], xp[:, HF:]
    A1, B1 = W1l[:, :HF].T, W1l[:, HF:].T
    C1, D1 = W1r[:, :HF].T, W1r[:, HF:].T
    A2, B2 = W2l[:, :HF].T, W2l[:, HF:].T
    C2, D2 = W2r[:, :HF].T, W2r[:, HF:].T
    b1 = b1l.reshape(1, CHN)
    b2 = b2l.reshape(1, CHN)

    agg1l, agg1r, deg0, deg1 = _sc_agg(True, nch)(xp, xp, eidx)
    deg0 = deg0.reshape(NP, 1)
    deg1 = deg1.reshape(NP, 1)
    hl, hr = _tc_layer(True, True)(agg1l, agg1r, xl, xr, deg0, deg1,
                                   A1, B1, C1, D1, b1)
    agg2l, agg2r = _sc_agg(False, nch)(xp, xp, eidx)
    out = _tc_layer(False, False)(agg2l, agg2r, hl, hr, deg0, deg1,
                                  A2, B2, C2, D2, b2)
    return out[:N]


# speedup vs baseline: 1.2037x; 1.2037x over previous
import functools

import jax
import jax.numpy as jnp
from jax import lax
from jax.experimental import pallas as pl
from jax.experimental.pallas import tpu as pltpu
from jax.experimental.pallas import tpu_sc as plsc

N = 10000
NP = 10240
NH = NP // 2
NL = NH + 256
CHN = 256
CS = 64
RCS = 128
N_SUB = 16
RPT = NL // N_SUB
WPT = NH // N_SUB


@functools.cache
def _sc_route(nch0: int):
    ept = nch0 * RCS
    mesh = plsc.VectorSubcoreMesh(core_axis_name="c", subcore_axis_name="s",
                                  num_cores=2, num_subcores=N_SUB)
    out_type = [
        jax.ShapeDtypeStruct((2, N_SUB, 2, ept), jnp.int32),
        jax.ShapeDtypeStruct((2, N_SUB, 16), jnp.int32),
    ]
    scratch = [
        pltpu.VMEM((nch0, 2, RCS), jnp.int32),
        pltpu.VMEM((ept + 80,), jnp.int32),
        pltpu.VMEM((ept + 80,), jnp.int32),
        pltpu.VMEM((16,), jnp.int32),
    ]

    def body(eidx_h, routed, counts, idxb, souts, douts, cntv):
        cid = lax.axis_index("c")
        sid = lax.axis_index("s")
        lo = cid * NH
        pltpu.sync_copy(eidx_h.at[sid], idxb)

        trash = jnp.int32(ept + 64) + lax.iota(jnp.int32, 16)

        def grp(g, ptr):
            j = g // (RCS // 16)
            v = (g % (RCS // 16)) * 16
            s16 = idxb[j, 0, pl.ds(v, 16)]
            d16 = idxb[j, 1, pl.ds(v, 16)]
            dl = d16 - lo
            mask = (dl >= 0) & (dl < NH)
            inc = jnp.where(mask, jnp.int32(1), jnp.int32(0))
            pc = jnp.cumsum(inc)
            pos = jnp.where(mask, ptr + pc - 1, trash)
            plsc.store_scatter(souts, [pos], s16)
            plsc.store_scatter(douts, [pos], dl)
            return ptr + jnp.max(pc)
        ptr = lax.fori_loop(0, ept // 16, grp, jnp.int32(0))

        for g in range(CS // 16):
            souts[pl.ds(ptr + g * 16, 16)] = jnp.zeros((16,), jnp.int32)
            douts[pl.ds(ptr + g * 16, 16)] = (
                lax.iota(jnp.int32, 16) + jnp.int32(NH + g * 16))
        nchunks = lax.div(ptr + CS - 1, jnp.int32(CS))
        cntv[pl.ds(0, 16)] = jnp.zeros((16,), jnp.int32) + nchunks

        pltpu.sync_copy(souts.at[pl.ds(0, ept)], routed.at[cid, sid, 0])
        pltpu.sync_copy(douts.at[pl.ds(0, ept)], routed.at[cid, sid, 1])
        pltpu.sync_copy(cntv, counts.at[cid, sid])

    return pl.kernel(
        body, out_type=out_type, mesh=mesh, scratch_types=scratch,
        compiler_params=pltpu.CompilerParams(needs_layout_passes=False))


@functools.cache
def _sc_agg(compute_deg: bool, ept: int):
    nch = ept // CS
    assert nch % 4 == 0 and nch >= 12
    mesh = plsc.VectorSubcoreMesh(core_axis_name="c", subcore_axis_name="s",
                                  num_cores=2, num_subcores=N_SUB)
    out_type = [jax.ShapeDtypeStruct((NP, 2, 128), jnp.float32)]
    scratch = (
        [pltpu.VMEM((2, CS), jnp.int32)] * 4
        + [pltpu.VMEM((CS, 2, 128), jnp.float32)] * 2
        + [pltpu.VMEM((16,), jnp.int32)]
        + [pltpu.VMEM_SHARED((NL, 2, 128), jnp.float32)]
        + [pltpu.SemaphoreType.DMA] * 8
    )
    if compute_deg:
        out_type.append(jax.ShapeDtypeStruct((NP,), jnp.float32))
        scratch += [
            pltpu.VMEM((CS,), jnp.float32),
            pltpu.VMEM((640,), jnp.float32),
            pltpu.VMEM_SHARED((NL,), jnp.float32),
        ]

    def body(tbl, routed, counts, out, *rest):
        if compute_deg:
            (deg_out, i0, i1, i2, i3, r0, r1, cntv, acc,
             m0, m1, m2, m3, g0, g1, s0, s1, ones_v, zdeg_v, dacc) = rest
        else:
            (i0, i1, i2, i3, r0, r1, cntv, acc,
             m0, m1, m2, m3, g0, g1, s0, s1) = rest
        islot = (i0, i1, i2, i3)
        isem = (m0, m1, m2, m3)
        rows = (r0, r1)
        gsem = (g0, g1)
        ssem = (s0, s1)
        cid = lax.axis_index("c")
        sid = lax.axis_index("s")

        pltpu.sync_copy(counts.at[cid, sid], cntv)
        cnt = cntv[pl.ds(0, 16)][0]

        def zrow(i, _):
            r0[i // 16, (i % 16) // 8, pl.ds((i % 8) * 16, 16)] = (
                jnp.zeros((16,), jnp.float32))
            return 0
        lax.fori_loop(0, CS * (CHN // 16), zrow, 0)
        zb = sid * RPT
        for q in range(RPT // CS):
            pltpu.sync_copy(r0, acc.at[pl.ds(zb + q * CS, CS)])
        rem = RPT % CS
        if rem:
            pltpu.sync_copy(r0.at[pl.ds(0, rem)],
                            acc.at[pl.ds(zb + (RPT // CS) * CS, rem)])

        if compute_deg:
            def fill_ones(i, _):
                ones_v[pl.ds(i * 16, 16)] = jnp.ones((16,), jnp.float32)
                return 0
            lax.fori_loop(0, CS // 16, fill_ones, 0)

            def zdeg(i, _):
                zdeg_v[pl.ds(i * 16, 16)] = jnp.zeros((16,), jnp.float32)
                return 0
            lax.fori_loop(0, 640 // 16, zdeg, 0)
            @pl.when(sid < 8)
            def _():
                pltpu.sync_copy(zdeg_v, dacc.at[pl.ds(sid * 640, 640)])

            @pl.when(sid == 8)
            def _():
                pltpu.sync_copy(zdeg_v.at[pl.ds(0, NL - 8 * 640)],
                                dacc.at[pl.ds(8 * 640, NL - 8 * 640)])

        plsc.subcore_barrier()

        def ifire(j, k):
            @pl.when(j < cnt)
            def _():
                pltpu.async_copy(routed.at[cid, sid, 0, pl.ds(j * CS, CS)],
                                 islot[k].at[0], isem[k])
                pltpu.async_copy(routed.at[cid, sid, 1, pl.ds(j * CS, CS)],
                                 islot[k].at[1], isem[k])

        def iwait(j, k):
            @pl.when(j < cnt)
            def _():
                pltpu.make_async_copy(
                    routed.at[cid, sid, 0, pl.ds(j * CS, CS)],
                    islot[k].at[0], isem[k]).wait()
                pltpu.make_async_copy(
                    routed.at[cid, sid, 1, pl.ds(j * CS, CS)],
                    islot[k].at[1], isem[k]).wait()

        def gfire(j, b, k):
            @pl.when(j < cnt)
            def _():
                pltpu.async_copy(tbl.at[islot[k].at[0]], rows[b], gsem[b])

        def gwait(j, b, k):
            @pl.when(j < cnt)
            def _():
                pltpu.make_async_copy(tbl.at[islot[k].at[0]], rows[b],
                                      gsem[b]).wait()

        def sfire(j, b, k):
            @pl.when(j < cnt)
            def _():
                pltpu.async_copy(rows[b], acc.at[islot[k].at[1]], ssem[b],
                                 add=True)

        def swait(j, b, k):
            @pl.when(j < cnt)
            def _():
                pltpu.make_async_copy(rows[b], acc.at[islot[k].at[1]],
                                      ssem[b]).wait()

        def step(j, b, k, fire_i=True, fire_g=True, first=False):
            if fire_g:
                iwait(j + 1, (k + 1) % 4)
            if not first:
                swait(j - 1, 1 - b, (k - 1) % 4)
            if fire_g:
                gfire(j + 1, 1 - b, (k + 1) % 4)
            if fire_i:
                ifire(j + 3, (k + 3) % 4)
            gwait(j, b, k)
            sfire(j, b, k)
            if compute_deg:
                @pl.when(j < cnt)
                def _():
                    pltpu.sync_copy(ones_v, dacc.at[islot[k].at[1]], add=True)

        ifire(0, 0)
        ifire(1, 1)
        ifire(2, 2)
        iwait(0, 0)
        gfire(0, 0, 0)
        step(0, 0, 0, first=True)
        step(1, 1, 1)
        step(2, 0, 2)
        step(3, 1, 3)

        def obody(o, _):
            j0 = o * 4 + 4
            for t in range(4):
                step(j0 + t, t % 2, t)
            return 0
        lax.fori_loop(0, (nch - 8) // 4, obody, 0)

        step(nch - 4, 0, 0)
        step(nch - 3, 1, 1, fire_i=False)
        step(nch - 2, 0, 2, fire_i=False)
        step(nch - 1, 1, 3, fire_i=False, fire_g=False)
        swait(nch - 1, 1, 3)

        plsc.subcore_barrier()

        wb = sid * WPT
        pltpu.sync_copy(acc.at[pl.ds(wb, WPT)],
                        out.at[pl.ds(cid * NH + wb, WPT)])
        if compute_deg:
            @pl.when(sid < 8)
            def _():
                pltpu.sync_copy(dacc.at[pl.ds(sid * 640, 640)],
                                deg_out.at[pl.ds(cid * NH + sid * 640, 640)])

    return pl.kernel(body, out_type=out_type, mesh=mesh, scratch_types=scratch)


@functools.cache
def _tc_layer(relu: bool):
    BLK = 1024

    def body(agg, x, deg, wl, wr, b, out):
        d = jnp.maximum(deg[...], 1.0)
        acc = jnp.dot(agg[...] / d, wl[...], preferred_element_type=jnp.float32)
        acc = acc + jnp.dot(x[...], wr[...], preferred_element_type=jnp.float32)
        acc = acc + b[...]
        if relu:
            acc = jnp.maximum(acc, 0.0)
        out[...] = acc

    row = lambda i: (i, 0)
    full = lambda i: (0, 0)
    in_specs = (
        [pl.BlockSpec((BLK, CHN), row)] * 2
        + [pl.BlockSpec((BLK, 1), row)]
        + [pl.BlockSpec((CHN, CHN), full)] * 2
        + [pl.BlockSpec((1, CHN), full)]
    )
    return pl.pallas_call(
        body, grid=(NP // BLK,), in_specs=in_specs,
        out_specs=pl.BlockSpec((BLK, CHN), row),
        out_shape=jax.ShapeDtypeStruct((NP, CHN), jnp.float32))


def kernel(x, edge_index, W1l, b1l, W1r, W2l, b2l, W2r):
    x = x.astype(jnp.float32)
    src = edge_index[0].astype(jnp.int32)
    dst = edge_index[1].astype(jnp.int32)
    e = src.shape[0]
    nch0 = -(-e // (N_SUB * RCS * 2)) * 2
    pad = nch0 * N_SUB * RCS - e
    srcp = jnp.concatenate(
        [src, jnp.zeros((pad,), jnp.int32)]).reshape(N_SUB, nch0, RCS)
    dstp = jnp.concatenate(
        [dst, N + (jnp.arange(pad, dtype=jnp.int32) % (NP - N))]
    ).reshape(N_SUB, nch0, RCS)
    eidx = jnp.stack([srcp, dstp], axis=2)
    xp = jnp.pad(x, ((0, NP - N), (0, 0)))
    w1l = W1l.T
    w1r = W1r.T
    w2l = W2l.T
    w2r = W2r.T
    b1 = b1l.reshape(1, CHN)
    b2 = b2l.reshape(1, CHN)

    ept = nch0 * RCS
    routed, counts = _sc_route(nch0)(eidx)
    agg1, deg = _sc_agg(True, ept)(xp.reshape(NP, 2, 128), routed, counts)
    deg2 = deg.reshape(NP, 1)
    h = _tc_layer(True)(agg1.reshape(NP, CHN), xp, deg2, w1l, w1r, b1)
    agg2, = _sc_agg(False, ept)(h.reshape(NP, 2, 128), routed, counts)
    out = _tc_layer(False)(agg2.reshape(NP, CHN), h, deg2, w2l, w2r, b2)
    return out[:N]


# speedup vs baseline: 2.6285x; 2.1836x over previous
import functools

import jax
import jax.numpy as jnp
from jax import lax
from jax.experimental import pallas as pl
from jax.experimental.pallas import tpu as pltpu
from jax.experimental.pallas import tpu_sc as plsc

N = 10000
NP = 10240
NH = NP // 2
NL = NH + 256
CHN = 256
CS = 64
RCS = 128
N_SUB = 16
RPT = NL // N_SUB
WPT = NH // N_SUB


@functools.cache
def _sc_route(nch0: int):
    ept = nch0 * RCS
    mesh = plsc.VectorSubcoreMesh(core_axis_name="c", subcore_axis_name="s",
                                  num_cores=2, num_subcores=N_SUB)
    out_type = [
        jax.ShapeDtypeStruct((2, N_SUB, 2, ept), jnp.int32),
        jax.ShapeDtypeStruct((2, N_SUB, 16), jnp.int32),
    ]
    scratch = [
        pltpu.VMEM((nch0, 2, RCS), jnp.int32),
        pltpu.VMEM((ept + 80,), jnp.int32),
        pltpu.VMEM((ept + 80,), jnp.int32),
        pltpu.VMEM((16,), jnp.int32),
    ]

    def body(eidx_h, routed, counts, idxb, souts, douts, cntv):
        cid = lax.axis_index("c")
        sid = lax.axis_index("s")
        lo = cid * NH
        pltpu.sync_copy(eidx_h.at[sid], idxb)

        trash = jnp.int32(ept + 64) + lax.iota(jnp.int32, 16)

        def grp(g, ptr):
            j = g // (RCS // 16)
            v = (g % (RCS // 16)) * 16
            s16 = idxb[j, 0, pl.ds(v, 16)]
            d16 = idxb[j, 1, pl.ds(v, 16)]
            dl = d16 - lo
            mask = (dl >= 0) & (dl < NH)
            inc = jnp.where(mask, jnp.int32(1), jnp.int32(0))
            pc = jnp.cumsum(inc)
            pos = jnp.where(mask, ptr + pc - 1, trash)
            plsc.store_scatter(souts, [pos], s16)
            plsc.store_scatter(douts, [pos], dl)
            return ptr + jnp.max(pc)
        ptr = lax.fori_loop(0, ept // 16, grp, jnp.int32(0))

        for g in range(CS // 16):
            souts[pl.ds(ptr + g * 16, 16)] = (
                lax.iota(jnp.int32, 16) * 61 + jnp.int32(g * 977))
            douts[pl.ds(ptr + g * 16, 16)] = (
                lax.iota(jnp.int32, 16) + jnp.int32(NH + g * 16))
        nchunks = lax.div(ptr + CS - 1, jnp.int32(CS))
        cntv[pl.ds(0, 16)] = jnp.zeros((16,), jnp.int32) + nchunks

        pltpu.sync_copy(souts.at[pl.ds(0, ept)], routed.at[cid, sid, 0])
        pltpu.sync_copy(douts.at[pl.ds(0, ept)], routed.at[cid, sid, 1])
        pltpu.sync_copy(cntv, counts.at[cid, sid])

    return pl.kernel(
        body, out_type=out_type, mesh=mesh, scratch_types=scratch,
        compiler_params=pltpu.CompilerParams(needs_layout_passes=False))


@functools.cache
def _sc_agg(compute_deg: bool, ept: int):
    nch = ept // CS
    assert nch % 4 == 0 and nch >= 12
    mesh = plsc.VectorSubcoreMesh(core_axis_name="c", subcore_axis_name="s",
                                  num_cores=2, num_subcores=N_SUB)
    out_type = [jax.ShapeDtypeStruct((NP, 2, 128), jnp.float32)]
    scratch = (
        [pltpu.VMEM((2, CS), jnp.int32)] * 4
        + [pltpu.VMEM((CS, 2, 128), jnp.float32)] * 2
        + [pltpu.VMEM((16,), jnp.int32)]
        + [pltpu.VMEM_SHARED((NL, 2, 128), jnp.float32)]
        + [pltpu.SemaphoreType.DMA] * 8
    )
    if compute_deg:
        out_type.append(jax.ShapeDtypeStruct((NP,), jnp.float32))
        scratch += [
            pltpu.VMEM((CS,), jnp.float32),
            pltpu.VMEM((640,), jnp.float32),
            pltpu.VMEM_SHARED((NL,), jnp.float32),
        ]

    def body(tbl, routed, counts, out, *rest):
        if compute_deg:
            (deg_out, i0, i1, i2, i3, r0, r1, cntv, acc,
             m0, m1, m2, m3, g0, g1, s0, s1, ones_v, zdeg_v, dacc) = rest
        else:
            (i0, i1, i2, i3, r0, r1, cntv, acc,
             m0, m1, m2, m3, g0, g1, s0, s1) = rest
        islot = (i0, i1, i2, i3)
        isem = (m0, m1, m2, m3)
        rows = (r0, r1)
        gsem = (g0, g1)
        ssem = (s0, s1)
        cid = lax.axis_index("c")
        sid = lax.axis_index("s")

        pltpu.sync_copy(counts.at[cid, sid], cntv)
        cnt = cntv[pl.ds(0, 16)][0]

        def zrow(i, _):
            r0[i // 16, (i % 16) // 8, pl.ds((i % 8) * 16, 16)] = (
                jnp.zeros((16,), jnp.float32))
            return 0
        lax.fori_loop(0, CS * (CHN // 16), zrow, 0)
        zb = sid * RPT
        for q in range(RPT // CS):
            pltpu.sync_copy(r0, acc.at[pl.ds(zb + q * CS, CS)])
        rem = RPT % CS
        if rem:
            pltpu.sync_copy(r0.at[pl.ds(0, rem)],
                            acc.at[pl.ds(zb + (RPT // CS) * CS, rem)])

        if compute_deg:
            def fill_ones(i, _):
                ones_v[pl.ds(i * 16, 16)] = jnp.ones((16,), jnp.float32)
                return 0
            lax.fori_loop(0, CS // 16, fill_ones, 0)

            def zdeg(i, _):
                zdeg_v[pl.ds(i * 16, 16)] = jnp.zeros((16,), jnp.float32)
                return 0
            lax.fori_loop(0, 640 // 16, zdeg, 0)
            @pl.when(sid < 8)
            def _():
                pltpu.sync_copy(zdeg_v, dacc.at[pl.ds(sid * 640, 640)])

            @pl.when(sid == 8)
            def _():
                pltpu.sync_copy(zdeg_v.at[pl.ds(0, NL - 8 * 640)],
                                dacc.at[pl.ds(8 * 640, NL - 8 * 640)])

        plsc.subcore_barrier()

        def ifire(j, k):
            @pl.when(j < cnt)
            def _():
                pltpu.async_copy(routed.at[cid, sid, 0, pl.ds(j * CS, CS)],
                                 islot[k].at[0], isem[k])
                pltpu.async_copy(routed.at[cid, sid, 1, pl.ds(j * CS, CS)],
                                 islot[k].at[1], isem[k])

        def iwait(j, k):
            @pl.when(j < cnt)
            def _():
                pltpu.make_async_copy(
                    routed.at[cid, sid, 0, pl.ds(j * CS, CS)],
                    islot[k].at[0], isem[k]).wait()
                pltpu.make_async_copy(
                    routed.at[cid, sid, 1, pl.ds(j * CS, CS)],
                    islot[k].at[1], isem[k]).wait()

        def gfire(j, b, k):
            @pl.when(j < cnt)
            def _():
                pltpu.async_copy(tbl.at[islot[k].at[0]], rows[b], gsem[b])

        def gwait(j, b, k):
            @pl.when(j < cnt)
            def _():
                pltpu.make_async_copy(tbl.at[islot[k].at[0]], rows[b],
                                      gsem[b]).wait()

        def sfire(j, b, k):
            @pl.when(j < cnt)
            def _():
                pltpu.async_copy(rows[b], acc.at[islot[k].at[1]], ssem[b],
                                 add=True)

        def swait(j, b, k):
            @pl.when(j < cnt)
            def _():
                pltpu.make_async_copy(rows[b], acc.at[islot[k].at[1]],
                                      ssem[b]).wait()

        def step(j, b, k, fire_i=True, fire_g=True, first=False):
            if fire_g:
                iwait(j + 1, (k + 1) % 4)
            if not first:
                swait(j - 1, 1 - b, (k - 1) % 4)
            if fire_g:
                gfire(j + 1, 1 - b, (k + 1) % 4)
            if fire_i:
                ifire(j + 3, (k + 3) % 4)
            gwait(j, b, k)
            sfire(j, b, k)
            if compute_deg:
                @pl.when(j < cnt)
                def _():
                    pltpu.sync_copy(ones_v, dacc.at[islot[k].at[1]], add=True)

        ifire(0, 0)
        ifire(1, 1)
        ifire(2, 2)
        iwait(0, 0)
        gfire(0, 0, 0)
        step(0, 0, 0, first=True)
        step(1, 1, 1)
        step(2, 0, 2)
        step(3, 1, 3)

        def obody(o, _):
            j0 = o * 4 + 4
            for t in range(4):
                step(j0 + t, t % 2, t)
            return 0
        lax.fori_loop(0, (nch - 8) // 4, obody, 0)

        step(nch - 4, 0, 0)
        step(nch - 3, 1, 1, fire_i=False)
        step(nch - 2, 0, 2, fire_i=False)
        step(nch - 1, 1, 3, fire_i=False, fire_g=False)
        swait(nch - 1, 1, 3)

        plsc.subcore_barrier()

        wb = sid * WPT
        pltpu.sync_copy(acc.at[pl.ds(wb, WPT)],
                        out.at[pl.ds(cid * NH + wb, WPT)])
        if compute_deg:
            @pl.when(sid < 8)
            def _():
                pltpu.sync_copy(dacc.at[pl.ds(sid * 640, 640)],
                                deg_out.at[pl.ds(cid * NH + sid * 640, 640)])

    return pl.kernel(body, out_type=out_type, mesh=mesh, scratch_types=scratch)


@functools.cache
def _tc_layer(relu: bool):
    BLK = 1024

    def body(agg, x, deg, wl, wr, b, out):
        d = jnp.maximum(deg[...], 1.0)
        acc = jnp.dot(agg[...] / d, wl[...], preferred_element_type=jnp.float32)
        acc = acc + jnp.dot(x[...], wr[...], preferred_element_type=jnp.float32)
        acc = acc + b[...]
        if relu:
            acc = jnp.maximum(acc, 0.0)
        out[...] = acc

    row = lambda i: (i, 0)
    full = lambda i: (0, 0)
    in_specs = (
        [pl.BlockSpec((BLK, CHN), row)] * 2
        + [pl.BlockSpec((BLK, 1), row)]
        + [pl.BlockSpec((CHN, CHN), full)] * 2
        + [pl.BlockSpec((1, CHN), full)]
    )
    return pl.pallas_call(
        body, grid=(NP // BLK,), in_specs=in_specs,
        out_specs=pl.BlockSpec((BLK, CHN), row),
        out_shape=jax.ShapeDtypeStruct((NP, CHN), jnp.float32))


def kernel(x, edge_index, W1l, b1l, W1r, W2l, b2l, W2r):
    x = x.astype(jnp.float32)
    src = edge_index[0].astype(jnp.int32)
    dst = edge_index[1].astype(jnp.int32)
    e = src.shape[0]
    nch0 = -(-e // (N_SUB * RCS * 2)) * 2
    pad = nch0 * N_SUB * RCS - e
    srcp = jnp.concatenate(
        [src, (jnp.arange(pad, dtype=jnp.int32) * 997) % N]
    ).reshape(N_SUB, nch0, RCS)
    dstp = jnp.concatenate(
        [dst, N + (jnp.arange(pad, dtype=jnp.int32) % (NP - N))]
    ).reshape(N_SUB, nch0, RCS)
    eidx = jnp.stack([srcp, dstp], axis=2)
    xp = jnp.pad(x, ((0, NP - N), (0, 0)))
    w1l = W1l.T
    w1r = W1r.T
    w2l = W2l.T
    w2r = W2r.T
    b1 = b1l.reshape(1, CHN)
    b2 = b2l.reshape(1, CHN)

    ept = nch0 * RCS
    routed, counts = _sc_route(nch0)(eidx)
    agg1, deg = _sc_agg(True, ept)(xp.reshape(NP, 2, 128), routed, counts)
    deg2 = deg.reshape(NP, 1)
    h = _tc_layer(True)(agg1.reshape(NP, CHN), xp, deg2, w1l, w1r, b1)
    agg2, = _sc_agg(False, ept)(h.reshape(NP, 2, 128), routed, counts)
    out = _tc_layer(False)(agg2.reshape(NP, CHN), h, deg2, w2l, w2r, b2)
    return out[:N]


# speedup vs baseline: 2.8095x; 1.0689x over previous
import functools

import jax
import jax.numpy as jnp
from jax import lax
from jax.experimental import pallas as pl
from jax.experimental.pallas import tpu as pltpu
from jax.experimental.pallas import tpu_sc as plsc

N = 10000
NP = 10240
NH = NP // 2
NL = NH + 256
CHN = 256
CS = 64
RCS = 128
N_SUB = 16
RPT = NL // N_SUB
WPT = NH // N_SUB


@functools.cache
def _sc_route(nch0: int):
    ept = nch0 * RCS
    mesh = plsc.VectorSubcoreMesh(core_axis_name="c", subcore_axis_name="s",
                                  num_cores=2, num_subcores=N_SUB)
    out_type = [
        jax.ShapeDtypeStruct((2, N_SUB, 2, ept), jnp.int32),
        jax.ShapeDtypeStruct((2, N_SUB, 16), jnp.int32),
    ]
    scratch = [
        pltpu.VMEM((nch0, 2, RCS), jnp.int32),
        pltpu.VMEM((ept + 80,), jnp.int32),
        pltpu.VMEM((ept + 80,), jnp.int32),
        pltpu.VMEM((16,), jnp.int32),
    ]

    def body(eidx_h, routed, counts, idxb, souts, douts, cntv):
        cid = lax.axis_index("c")
        sid = lax.axis_index("s")
        lo = cid * NH
        pltpu.sync_copy(eidx_h.at[sid], idxb)

        trash = jnp.int32(ept + 64) + lax.iota(jnp.int32, 16)

        def grp(g, ptr):
            j = g // (RCS // 16)
            v = (g % (RCS // 16)) * 16
            s16 = idxb[j, 0, pl.ds(v, 16)]
            d16 = idxb[j, 1, pl.ds(v, 16)]
            dl = d16 - lo
            mask = (dl >= 0) & (dl < NH)
            inc = jnp.where(mask, jnp.int32(1), jnp.int32(0))
            pc = jnp.cumsum(inc)
            pos = jnp.where(mask, ptr + pc - 1, trash)
            plsc.store_scatter(souts, [pos], s16)
            plsc.store_scatter(douts, [pos], dl)
            return ptr + jnp.max(pc)
        ptr = lax.fori_loop(0, ept // 16, grp, jnp.int32(0))

        for g in range(CS // 16):
            souts[pl.ds(ptr + g * 16, 16)] = (
                lax.iota(jnp.int32, 16) * 61 + jnp.int32(g * 977))
            douts[pl.ds(ptr + g * 16, 16)] = (
                lax.iota(jnp.int32, 16) + jnp.int32(NH + g * 16))
        nchunks = lax.div(ptr + CS - 1, jnp.int32(CS))
        cntv[pl.ds(0, 16)] = jnp.zeros((16,), jnp.int32) + nchunks

        pltpu.sync_copy(souts.at[pl.ds(0, ept)], routed.at[cid, sid, 0])
        pltpu.sync_copy(douts.at[pl.ds(0, ept)], routed.at[cid, sid, 1])
        pltpu.sync_copy(cntv, counts.at[cid, sid])

    return pl.kernel(
        body, out_type=out_type, mesh=mesh, scratch_types=scratch,
        compiler_params=pltpu.CompilerParams(needs_layout_passes=False))


@functools.cache
def _sc_agg(compute_deg: bool, ept: int):
    nch = ept // CS
    assert nch % 4 == 0 and nch >= 12
    mesh = plsc.VectorSubcoreMesh(core_axis_name="c", subcore_axis_name="s",
                                  num_cores=2, num_subcores=N_SUB)
    out_type = [jax.ShapeDtypeStruct((NP, 2, 128), jnp.float32)]
    scratch = (
        [pltpu.VMEM((2, CS), jnp.int32)] * 4
        + [pltpu.VMEM((CS, 2, 128), jnp.float32)] * 2
        + [pltpu.VMEM((16,), jnp.int32)]
        + [pltpu.VMEM_SHARED((NL, 2, 128), jnp.float32)]
        + [pltpu.SemaphoreType.DMA] * 8
    )
    if compute_deg:
        out_type.append(jax.ShapeDtypeStruct((NP,), jnp.float32))
        scratch += [
            pltpu.VMEM((CS,), jnp.float32),
            pltpu.VMEM((640,), jnp.float32),
            pltpu.VMEM_SHARED((NL,), jnp.float32),
        ]

    def body(tbl, routed, counts, out, *rest):
        if compute_deg:
            (deg_out, i0, i1, i2, i3, r0, r1, cntv, acc,
             m0, m1, m2, m3, g0, g1, s0, s1, ones_v, zdeg_v, dacc) = rest
        else:
            (i0, i1, i2, i3, r0, r1, cntv, acc,
             m0, m1, m2, m3, g0, g1, s0, s1) = rest
        islot = (i0, i1, i2, i3)
        isem = (m0, m1, m2, m3)
        rows = (r0, r1)
        gsem = (g0, g1)
        ssem = (s0, s1)
        cid = lax.axis_index("c")
        sid = lax.axis_index("s")

        pltpu.sync_copy(counts.at[cid, sid], cntv)
        cnt = cntv[pl.ds(0, 16)][0]

        def zrow(i, _):
            r0[i // 16, (i % 16) // 8, pl.ds((i % 8) * 16, 16)] = (
                jnp.zeros((16,), jnp.float32))
            return 0
        lax.fori_loop(0, CS * (CHN // 16), zrow, 0)
        zb = sid * RPT
        for q in range(RPT // CS):
            pltpu.sync_copy(r0, acc.at[pl.ds(zb + q * CS, CS)])
        rem = RPT % CS
        if rem:
            pltpu.sync_copy(r0.at[pl.ds(0, rem)],
                            acc.at[pl.ds(zb + (RPT // CS) * CS, rem)])

        if compute_deg:
            def fill_ones(i, _):
                ones_v[pl.ds(i * 16, 16)] = jnp.ones((16,), jnp.float32)
                return 0
            lax.fori_loop(0, CS // 16, fill_ones, 0)

            def zdeg(i, _):
                zdeg_v[pl.ds(i * 16, 16)] = jnp.zeros((16,), jnp.float32)
                return 0
            lax.fori_loop(0, 640 // 16, zdeg, 0)
            @pl.when(sid < 8)
            def _():
                pltpu.sync_copy(zdeg_v, dacc.at[pl.ds(sid * 640, 640)])

            @pl.when(sid == 8)
            def _():
                pltpu.sync_copy(zdeg_v.at[pl.ds(0, NL - 8 * 640)],
                                dacc.at[pl.ds(8 * 640, NL - 8 * 640)])

        plsc.subcore_barrier()

        def ifire(j, k):
            @pl.when(j < cnt)
            def _():
                pltpu.async_copy(routed.at[cid, sid, 0, pl.ds(j * CS, CS)],
                                 islot[k].at[0], isem[k])
                pltpu.async_copy(routed.at[cid, sid, 1, pl.ds(j * CS, CS)],
                                 islot[k].at[1], isem[k])

        def iwait(j, k):
            @pl.when(j < cnt)
            def _():
                pltpu.make_async_copy(
                    routed.at[cid, sid, 0, pl.ds(j * CS, CS)],
                    islot[k].at[0], isem[k]).wait()
                pltpu.make_async_copy(
                    routed.at[cid, sid, 1, pl.ds(j * CS, CS)],
                    islot[k].at[1], isem[k]).wait()

        def gfire(j, b, k):
            @pl.when(j < cnt)
            def _():
                pltpu.async_copy(tbl.at[islot[k].at[0]], rows[b], gsem[b])

        def gwait(j, b, k):
            @pl.when(j < cnt)
            def _():
                pltpu.make_async_copy(tbl.at[islot[k].at[0]], rows[b],
                                      gsem[b]).wait()

        def sfire(j, b, k):
            @pl.when(j < cnt)
            def _():
                pltpu.async_copy(rows[b], acc.at[islot[k].at[1]], ssem[b],
                                 add=True)

        def swait(j, b, k):
            @pl.when(j < cnt)
            def _():
                pltpu.make_async_copy(rows[b], acc.at[islot[k].at[1]],
                                      ssem[b]).wait()

        def step(j, b, k, fire_i=True, fire_g=True, first=False):
            if fire_g:
                iwait(j + 1, (k + 1) % 4)
            if not first:
                swait(j - 1, 1 - b, (k - 1) % 4)
            if fire_g:
                gfire(j + 1, 1 - b, (k + 1) % 4)
            if fire_i:
                ifire(j + 3, (k + 3) % 4)
            gwait(j, b, k)
            sfire(j, b, k)
            if compute_deg:
                @pl.when(j < cnt)
                def _():
                    pltpu.sync_copy(ones_v, dacc.at[islot[k].at[1]], add=True)

        ifire(0, 0)
        ifire(1, 1)
        ifire(2, 2)
        iwait(0, 0)
        gfire(0, 0, 0)
        step(0, 0, 0, first=True)
        step(1, 1, 1)
        step(2, 0, 2)
        step(3, 1, 3)

        def obody(o, _):
            j0 = o * 4 + 4
            for t in range(4):
                step(j0 + t, t % 2, t)
            return 0
        lax.fori_loop(0, (nch - 8) // 4, obody, 0)

        step(nch - 4, 0, 0)
        step(nch - 3, 1, 1, fire_i=False)
        step(nch - 2, 0, 2, fire_i=False)
        step(nch - 1, 1, 3, fire_i=False, fire_g=False)
        swait(nch - 1, 1, 3)

        plsc.subcore_barrier()

        wb = sid * WPT
        pltpu.sync_copy(acc.at[pl.ds(wb, WPT)],
                        out.at[pl.ds(cid * NH + wb, WPT)])
        if compute_deg:
            @pl.when(sid < 8)
            def _():
                pltpu.sync_copy(dacc.at[pl.ds(sid * 640, 640)],
                                deg_out.at[pl.ds(cid * NH + sid * 640, 640)])

    return pl.kernel(body, out_type=out_type, mesh=mesh, scratch_types=scratch)


BLK = 1024
_row2 = lambda i: (i, 0)
_row3 = lambda i: (i, 0, 0)
_full = lambda i: (0, 0)


def _tc_pre():
    def body(x3, c, d, b, out):
        acc = jnp.dot(x3[:, 0, :], c[...], preferred_element_type=jnp.float32)
        acc = acc + jnp.dot(x3[:, 1, :], d[...],
                            preferred_element_type=jnp.float32)
        out[...] = acc + b[...]

    in_specs = (
        [pl.BlockSpec((BLK, 2, 128), _row3)]
        + [pl.BlockSpec((128, CHN), _full)] * 2
        + [pl.BlockSpec((1, CHN), _full)]
    )
    return pl.pallas_call(
        body, grid=(NP // BLK,), in_specs=in_specs,
        out_specs=pl.BlockSpec((BLK, CHN), _row2),
        out_shape=jax.ShapeDtypeStruct((NP, CHN), jnp.float32))


@functools.cache
def _tc_post(relu: bool, out3: bool):
    def body(agg3, pre, deg, a, b, out):
        dd = jnp.maximum(deg[...], 1.0)
        acc = jnp.dot(agg3[:, 0, :] / dd, a[...],
                      preferred_element_type=jnp.float32)
        acc = acc + jnp.dot(agg3[:, 1, :] / dd, b[...],
                            preferred_element_type=jnp.float32)
        acc = acc + pre[...]
        if relu:
            acc = jnp.maximum(acc, 0.0)
        if out3:
            out[:, 0, :] = acc[:, :128]
            out[:, 1, :] = acc[:, 128:]
        else:
            out[...] = acc

    in_specs = (
        [pl.BlockSpec((BLK, 2, 128), _row3)]
        + [pl.BlockSpec((BLK, CHN), _row2)]
        + [pl.BlockSpec((BLK, 1), _row2)]
        + [pl.BlockSpec((128, CHN), _full)] * 2
    )
    if out3:
        out_specs = pl.BlockSpec((BLK, 2, 128), _row3)
        out_shape = jax.ShapeDtypeStruct((NP, 2, 128), jnp.float32)
    else:
        out_specs = pl.BlockSpec((BLK, CHN), _row2)
        out_shape = jax.ShapeDtypeStruct((NP, CHN), jnp.float32)
    return pl.pallas_call(body, grid=(NP // BLK,), in_specs=in_specs,
                          out_specs=out_specs, out_shape=out_shape)


def kernel(x, edge_index, W1l, b1l, W1r, W2l, b2l, W2r):
    x = x.astype(jnp.float32)
    src = edge_index[0].astype(jnp.int32)
    dst = edge_index[1].astype(jnp.int32)
    e = src.shape[0]
    nch0 = -(-e // (N_SUB * RCS * 2)) * 2
    pad = nch0 * N_SUB * RCS - e
    srcp = jnp.concatenate(
        [src, (jnp.arange(pad, dtype=jnp.int32) * 997) % N]
    ).reshape(N_SUB, nch0, RCS)
    dstp = jnp.concatenate(
        [dst, N + (jnp.arange(pad, dtype=jnp.int32) % (NP - N))]
    ).reshape(N_SUB, nch0, RCS)
    eidx = jnp.stack([srcp, dstp], axis=2)
    xp3 = jnp.pad(x, ((0, NP - N), (0, 0))).reshape(NP, 2, 128)
    A1, B1 = W1l[:, :128].T, W1l[:, 128:].T
    C1, D1 = W1r[:, :128].T, W1r[:, 128:].T
    A2, B2 = W2l[:, :128].T, W2l[:, 128:].T
    C2, D2 = W2r[:, :128].T, W2r[:, 128:].T
    b1 = b1l.reshape(1, CHN)
    b2 = b2l.reshape(1, CHN)

    ept = nch0 * RCS
    routed, counts = _sc_route(nch0)(eidx)
    pre1 = _tc_pre()(xp3, C1, D1, b1)
    agg1, deg = _sc_agg(True, ept)(xp3, routed, counts)
    deg2 = deg.reshape(NP, 1)
    h3 = _tc_post(True, True)(agg1, pre1, deg2, A1, B1)
    pre2 = _tc_pre()(h3, C2, D2, b2)
    agg2, = _sc_agg(False, ept)(h3, routed, counts)
    out = _tc_post(False, False)(agg2, pre2, deg2, A2, B2)
    return out[:N]
